# baseline (speedup 1.0000x reference)
import functools

import jax
import jax.numpy as jnp
from jax import lax
from jax.experimental import pallas as pl
from jax.experimental.pallas import tpu as pltpu
from jax.experimental.pallas import tpu_sc as plsc

_NE = 1024
_D = 64
_CC = 0.25
_BM = 512


def _vq_body(z_ref, e_ref, enc_ref, idx_ref, loss_ref, perp_ref,
             se2_acc, cnt_acc, loss_acc):
    i = pl.program_id(0)
    nsteps = pl.num_programs(0)
    n_tok = nsteps * _BM

    @pl.when(i == 0)
    def _prologue():
        e = e_ref[...]
        se2_acc[...] = jnp.sum(e * e, axis=1)[None, :]

    z = z_ref[...]
    sz2 = jnp.sum(z * z, axis=1, keepdims=True)
    mm2 = jax.lax.dot_general(z + z, e_ref[...], (((1,), (1,)), ((), ())),
                              preferred_element_type=jnp.float32)
    d = (sz2 + se2_acc[...]) - mm2

    dmin = jnp.min(d, axis=1, keepdims=True)
    col = jax.lax.broadcasted_iota(jnp.int32, (_BM, _NE), 1)
    idx = jnp.min(jnp.where(d == dmin, col, _NE), axis=1, keepdims=True)
    idx_ref[...] = idx
    enc = (col == idx).astype(jnp.float32)
    enc_ref[...] = enc

    tile_loss = jnp.sum(dmin)
    ones = jnp.ones((1, _BM), jnp.float32)
    tile_cnt = jax.lax.dot_general(ones, enc, (((1,), (0,)), ((), ())),
                                   preferred_element_type=jnp.float32)

    @pl.when(i == 0)
    def _init():
        cnt_acc[...] = tile_cnt
        loss_acc[0, 0] = tile_loss

    @pl.when(i > 0)
    def _accum():
        cnt_acc[...] += tile_cnt
        loss_acc[0, 0] += tile_loss

    @pl.when(i == nsteps - 1)
    def _finalize():
        avg = cnt_acc[...] * (1.0 / n_tok)
        perp_ref[0, 0] = jnp.exp(-jnp.sum(avg * jnp.log(avg + 1e-10)))
        loss_ref[0, 0] = (1.0 + _CC) * loss_acc[0, 0] / (n_tok * _D)


def _tc_stage(z_flat, embeddings):
    n_tok = z_flat.shape[0]
    return pl.pallas_call(
        _vq_body,
        grid=(n_tok // _BM,),
        in_specs=[
            pl.BlockSpec((_BM, _D), lambda i: (i, 0)),
            pl.BlockSpec((_NE, _D), lambda i: (0, 0)),
        ],
        out_specs=[
            pl.BlockSpec((_BM, _NE), lambda i: (i, 0)),
            pl.BlockSpec((_BM, 1), lambda i: (i, 0)),
            pl.BlockSpec(memory_space=pltpu.SMEM),
            pl.BlockSpec(memory_space=pltpu.SMEM),
        ],
        out_shape=[
            jax.ShapeDtypeStruct((n_tok, _NE), jnp.float32),
            jax.ShapeDtypeStruct((n_tok, 1), jnp.int32),
            jax.ShapeDtypeStruct((1, 1), jnp.float32),
            jax.ShapeDtypeStruct((1, 1), jnp.float32),
        ],
        scratch_shapes=[
            pltpu.VMEM((1, _NE), jnp.float32),
            pltpu.VMEM((1, _NE), jnp.float32),
            pltpu.SMEM((1, 1), jnp.float32),
        ],
    )(z_flat, embeddings)


def _make_sc_gather(n_tok):
    info = plsc.get_sparse_core_info()
    nw = info.num_cores * info.num_subcores
    b_per_w = n_tok // nw
    mesh = plsc.VectorSubcoreMesh(core_axis_name="c", subcore_axis_name="s")

    @functools.partial(
        pl.kernel, mesh=mesh,
        compiler_params=pltpu.CompilerParams(use_tc_tiling_on_sc=False),
        out_type=jax.ShapeDtypeStruct((n_tok, _D), jnp.float32),
        scratch_types=[
            pltpu.VMEM((b_per_w,), jnp.int32),
            pltpu.VMEM((b_per_w, _D), jnp.float32),
            pltpu.SemaphoreType.DMA,
        ],
    )
    def _gather(table_hbm, idx_hbm, out_hbm, idx_v, rows_v, sem):
        wid = lax.axis_index("s") * info.num_cores + lax.axis_index("c")
        base = wid * b_per_w
        pltpu.sync_copy(idx_hbm.at[pl.ds(base, b_per_w)], idx_v)
        pltpu.async_copy(table_hbm.at[idx_v], rows_v, sem).wait()
        pltpu.sync_copy(rows_v, out_hbm.at[pl.ds(base, b_per_w)])

    return _gather


def kernel(z, embeddings):
    b, c, h, w = z.shape
    z_flat = jnp.transpose(z, (0, 2, 3, 1)).reshape(-1, _D)
    n_tok = z_flat.shape[0]

    enc, idx, loss, perp = _tc_stage(z_flat, embeddings)
    q = _make_sc_gather(n_tok)(embeddings, idx.reshape(n_tok))
    quantized = q.reshape(b, c, h, w)
    return (quantized, loss[0, 0], perp[0, 0], enc)


# speedup vs baseline: 1.1114x; 1.1114x over previous
import functools

import jax
import jax.numpy as jnp
from jax import lax
from jax.experimental import pallas as pl
from jax.experimental.pallas import tpu as pltpu
from jax.experimental.pallas import tpu_sc as plsc

_NE = 1024
_D = 64
_CC = 0.25
_BM = 2048
_H = 2


def _vq_body(z_ref, e_ref, enc_ref, idx_ref, loss_ref, perp_ref,
             se2_acc, cnt_acc, loss_acc):
    i = pl.program_id(0)
    nsteps = pl.num_programs(0)
    n_tok = nsteps * _BM

    @pl.when(i == 0)
    def _prologue():
        e = e_ref[...]
        se2_acc[...] = jnp.sum(e * e, axis=1)[None, :]

    se2 = se2_acc[...]
    bh = _BM // _H
    col = jax.lax.broadcasted_iota(jnp.int32, (bh, _NE), 1)
    ones = jnp.ones((1, bh), jnp.float32)
    tile_loss = 0.0
    tile_cnt = jnp.zeros((1, _NE), jnp.float32)
    for h in range(_H):
        rows = pl.ds(h * bh, bh)
        z = z_ref[rows, :]
        sz2 = jnp.sum(z * z, axis=1, keepdims=True)
        mm2 = jax.lax.dot_general(z + z, e_ref[...], (((1,), (1,)), ((), ())),
                                  preferred_element_type=jnp.float32)
        d = (sz2 + se2) - mm2
        dmin = jnp.min(d, axis=1, keepdims=True)
        idx = jnp.min(jnp.where(d == dmin, col, _NE), axis=1, keepdims=True)
        idx_ref[rows, :] = idx
        enc = (col == idx).astype(jnp.float32)
        enc_ref[rows, :] = enc
        tile_loss += jnp.sum(dmin)
        tile_cnt += jax.lax.dot_general(ones, enc, (((1,), (0,)), ((), ())),
                                        preferred_element_type=jnp.float32)

    @pl.when(i == 0)
    def _init():
        cnt_acc[...] = tile_cnt
        loss_acc[0, 0] = tile_loss

    @pl.when(i > 0)
    def _accum():
        cnt_acc[...] += tile_cnt
        loss_acc[0, 0] += tile_loss

    @pl.when(i == nsteps - 1)
    def _finalize():
        avg = cnt_acc[...] * (1.0 / n_tok)
        perp_ref[0, 0] = jnp.exp(-jnp.sum(avg * jnp.log(avg + 1e-10)))
        loss_ref[0, 0] = (1.0 + _CC) * loss_acc[0, 0] / (n_tok * _D)


def _tc_stage(z_flat, embeddings):
    n_tok = z_flat.shape[0]
    return pl.pallas_call(
        _vq_body,
        grid=(n_tok // _BM,),
        in_specs=[
            pl.BlockSpec((_BM, _D), lambda i: (i, 0)),
            pl.BlockSpec((_NE, _D), lambda i: (0, 0)),
        ],
        out_specs=[
            pl.BlockSpec((_BM, _NE), lambda i: (i, 0)),
            pl.BlockSpec((_BM, 1), lambda i: (i, 0)),
            pl.BlockSpec(memory_space=pltpu.SMEM),
            pl.BlockSpec(memory_space=pltpu.SMEM),
        ],
        out_shape=[
            jax.ShapeDtypeStruct((n_tok, _NE), jnp.float32),
            jax.ShapeDtypeStruct((n_tok, 1), jnp.int32),
            jax.ShapeDtypeStruct((1, 1), jnp.float32),
            jax.ShapeDtypeStruct((1, 1), jnp.float32),
        ],
        scratch_shapes=[
            pltpu.VMEM((1, _NE), jnp.float32),
            pltpu.VMEM((1, _NE), jnp.float32),
            pltpu.SMEM((1, 1), jnp.float32),
        ],
    )(z_flat, embeddings)


def _make_sc_gather(n_tok):
    info = plsc.get_sparse_core_info()
    nw = info.num_cores * info.num_subcores
    b_per_w = n_tok // nw
    mesh = plsc.VectorSubcoreMesh(core_axis_name="c", subcore_axis_name="s")

    @functools.partial(
        pl.kernel, mesh=mesh,
        compiler_params=pltpu.CompilerParams(use_tc_tiling_on_sc=False),
        out_type=jax.ShapeDtypeStruct((n_tok, _D), jnp.float32),
        scratch_types=[
            pltpu.VMEM((b_per_w,), jnp.int32),
            pltpu.VMEM((b_per_w, _D), jnp.float32),
            pltpu.SemaphoreType.DMA,
        ],
    )
    def _gather(table_hbm, idx_hbm, out_hbm, idx_v, rows_v, sem):
        wid = lax.axis_index("s") * info.num_cores + lax.axis_index("c")
        base = wid * b_per_w
        pltpu.sync_copy(idx_hbm.at[pl.ds(base, b_per_w)], idx_v)
        pltpu.async_copy(table_hbm.at[idx_v], rows_v, sem).wait()
        pltpu.sync_copy(rows_v, out_hbm.at[pl.ds(base, b_per_w)])

    return _gather


def kernel(z, embeddings):
    b, c, h, w = z.shape
    z_flat = jnp.transpose(z, (0, 2, 3, 1)).reshape(-1, _D)
    n_tok = z_flat.shape[0]

    enc, idx, loss, perp = _tc_stage(z_flat, embeddings)
    q = _make_sc_gather(n_tok)(embeddings, idx.reshape(n_tok))
    quantized = q.reshape(b, c, h, w)
    return (quantized, loss[0, 0], perp[0, 0], enc)
